# baseline (device time: 100858 ns/iter reference)
import jax
import jax.numpy as jnp
from jax import lax
from jax.experimental import pallas as pl
from jax.experimental.pallas import tpu as pltpu

N_DEV = 4
N_TOK = 2048
D_IN = 512
D_OUT = 1024
E_LOCAL = 4
CH = N_TOK // N_DEV
N_HOPS = 2 * (N_DEV - 1)


def kernel(x, router_W, route_idx, expert_W):
    del router_W

    def body(x_ref, idx_ref, w_ref, out_ref, send_buf, recv_buf, send_sems, recv_sems):
        my = lax.axis_index("i")
        left = lax.rem(my + N_DEV - 1, N_DEV)
        right = lax.rem(my + 1, N_DEV)

        barrier_sem = pltpu.get_barrier_semaphore()
        for nbr in (left, right):
            pl.semaphore_signal(
                barrier_sem, inc=1,
                device_id=(nbr,), device_id_type=pl.DeviceIdType.MESH,
            )
        pl.semaphore_wait(barrier_sem, 2)

        idx = idx_ref[...]
        xb = x_ref[...].astype(jnp.bfloat16)
        acc = jnp.zeros((N_TOK, D_OUT), jnp.float32)
        for j in range(E_LOCAL):
            ge = my * E_LOCAL + j
            xm = jnp.where(idx == ge, xb, jnp.bfloat16(0.0))
            acc = acc + lax.dot(
                xm, w_ref[j].astype(jnp.bfloat16),
                preferred_element_type=jnp.float32,
            )
        out_ref[...] = acc

        for s in range(N_DEV - 1):
            c_send = lax.rem(my + (N_DEV - s), N_DEV)
            send_buf[s % 2] = out_ref[pl.ds(c_send * CH, CH), :].astype(jnp.bfloat16)
            rdma = pltpu.make_async_remote_copy(
                src_ref=send_buf.at[s % 2],
                dst_ref=recv_buf.at[s],
                send_sem=send_sems.at[s],
                recv_sem=recv_sems.at[s],
                device_id=(right,),
                device_id_type=pl.DeviceIdType.MESH,
            )
            rdma.start()
            rdma.wait()
            c_recv = lax.rem(my + (N_DEV - s - 1), N_DEV)
            out_ref[pl.ds(c_recv * CH, CH), :] = (
                out_ref[pl.ds(c_recv * CH, CH), :]
                + recv_buf[s].astype(jnp.float32)
            )

        for t in range(N_DEV - 1):
            k = (N_DEV - 1) + t
            if t == 0:
                c_send = lax.rem(my + 1, N_DEV)
                send_buf[k % 2] = out_ref[pl.ds(c_send * CH, CH), :].astype(jnp.bfloat16)
                src = send_buf.at[k % 2]
            else:
                src = recv_buf.at[k - 1]
            rdma = pltpu.make_async_remote_copy(
                src_ref=src,
                dst_ref=recv_buf.at[k],
                send_sem=send_sems.at[k],
                recv_sem=recv_sems.at[k],
                device_id=(right,),
                device_id_type=pl.DeviceIdType.MESH,
            )
            rdma.start()
            rdma.wait()
            c_recv = lax.rem(my + (N_DEV - t), N_DEV)
            out_ref[pl.ds(c_recv * CH, CH), :] = recv_buf[k].astype(jnp.float32)

    return pl.pallas_call(
        body,
        out_shape=jax.ShapeDtypeStruct((N_TOK, D_OUT), jnp.float32),
        in_specs=[
            pl.BlockSpec(memory_space=pltpu.VMEM),
            pl.BlockSpec(memory_space=pltpu.VMEM),
            pl.BlockSpec(memory_space=pltpu.VMEM),
        ],
        out_specs=pl.BlockSpec(memory_space=pltpu.VMEM),
        scratch_shapes=[
            pltpu.VMEM((2, CH, D_OUT), jnp.bfloat16),
            pltpu.VMEM((N_HOPS, CH, D_OUT), jnp.bfloat16),
            pltpu.SemaphoreType.DMA((N_HOPS,)),
            pltpu.SemaphoreType.DMA((N_HOPS,)),
        ],
        compiler_params=pltpu.CompilerParams(collective_id=0),
    )(x, route_idx, expert_W)


# device time: 59854 ns/iter; 1.6851x vs baseline; 1.6851x over previous
import jax
import jax.numpy as jnp
from jax import lax
from jax.experimental import pallas as pl
from jax.experimental.pallas import tpu as pltpu

N_DEV = 4
N_TOK = 2048
D_IN = 512
D_OUT = 1024
E_LOCAL = 4
HALF = N_TOK // 2
CH = HALF // N_DEV
N_HOPS = 2 * (N_DEV - 1)
A, B = 0, 1


def kernel(x, router_W, route_idx, expert_W):
    del router_W

    def body(x_ref, idx_ref, w_ref, out_ref, wb_ref, stage_ref, recv_buf,
             send_sems, recv_sems):
        my = lax.axis_index("i")
        left = lax.rem(my + N_DEV - 1, N_DEV)
        right = lax.rem(my + 1, N_DEV)

        def m(k):
            return lax.rem(my + k, N_DEV)

        def row_a(c):
            return c * CH

        def row_b(c):
            return HALF + c * CH

        barrier_sem = pltpu.get_barrier_semaphore()
        for nbr in (left, right):
            pl.semaphore_signal(
                barrier_sem, inc=1,
                device_id=(nbr,), device_id_type=pl.DeviceIdType.MESH,
            )
        pl.semaphore_wait(barrier_sem, 2)

        for j in range(E_LOCAL):
            wb_ref[j] = w_ref[j].astype(jnp.bfloat16)

        def compute_chunk(row0):
            xb = x_ref[pl.ds(row0, CH), :].astype(jnp.bfloat16)
            idc = idx_ref[pl.ds(row0, CH), :]
            acc = jnp.zeros((CH, D_OUT), jnp.float32)
            for j in range(E_LOCAL):
                ge = my * E_LOCAL + j
                xm = jnp.where(idc == ge, xb, jnp.bfloat16(0.0))
                acc = acc + lax.dot(
                    xm, wb_ref[j], preferred_element_type=jnp.float32)
            out_ref[pl.ds(row0, CH), :] = acc

        def add_recv(ring, k, row0):
            out_ref[pl.ds(row0, CH), :] = (
                out_ref[pl.ds(row0, CH), :]
                + recv_buf[ring, k].astype(jnp.float32))

        dst_dev = {A: right, B: left}

        def stage_and_send(ring, k, slot, row0):
            stage_ref[ring, slot] = out_ref[pl.ds(row0, CH), :].astype(
                jnp.bfloat16)
            r = pltpu.make_async_remote_copy(
                src_ref=stage_ref.at[ring, slot],
                dst_ref=recv_buf.at[ring, k],
                send_sem=send_sems.at[ring, k],
                recv_sem=recv_sems.at[ring, k],
                device_id=(dst_dev[ring],),
                device_id_type=pl.DeviceIdType.MESH,
            )
            r.start()
            return r

        def forward(ring, k_src, k):
            r = pltpu.make_async_remote_copy(
                src_ref=recv_buf.at[ring, k_src],
                dst_ref=recv_buf.at[ring, k],
                send_sem=send_sems.at[ring, k],
                recv_sem=recv_sems.at[ring, k],
                device_id=(dst_dev[ring],),
                device_id_type=pl.DeviceIdType.MESH,
            )
            r.start()
            return r

        rd = {}

        compute_chunk(row_a(m(0)))
        compute_chunk(row_b(m(0)))
        rd[A, 0] = stage_and_send(A, 0, 0, row_a(m(0)))
        rd[B, 0] = stage_and_send(B, 0, 0, row_b(m(0)))
        compute_chunk(row_a(m(3)))
        compute_chunk(row_b(m(1)))

        rd[A, 0].wait_recv()
        add_recv(A, 0, row_a(m(3)))
        rd[A, 1] = stage_and_send(A, 1, 1, row_a(m(3)))
        rd[B, 0].wait_recv()
        add_recv(B, 0, row_b(m(1)))
        rd[B, 1] = stage_and_send(B, 1, 1, row_b(m(1)))
        compute_chunk(row_a(m(2)))
        compute_chunk(row_b(m(2)))

        rd[A, 0].wait_send()
        rd[A, 1].wait_recv()
        add_recv(A, 1, row_a(m(2)))
        rd[A, 2] = stage_and_send(A, 2, 0, row_a(m(2)))
        rd[B, 0].wait_send()
        rd[B, 1].wait_recv()
        add_recv(B, 1, row_b(m(2)))
        rd[B, 2] = stage_and_send(B, 2, 0, row_b(m(2)))
        compute_chunk(row_a(m(1)))
        compute_chunk(row_b(m(3)))

        rd[A, 2].wait_recv()
        add_recv(A, 2, row_a(m(1)))
        rd[A, 1].wait_send()
        rd[A, 3] = stage_and_send(A, 3, 1, row_a(m(1)))
        rd[B, 2].wait_recv()
        add_recv(B, 2, row_b(m(3)))
        rd[B, 1].wait_send()
        rd[B, 3] = stage_and_send(B, 3, 1, row_b(m(3)))

        rd[A, 3].wait_recv()
        rd[A, 4] = forward(A, 3, 4)
        out_ref[pl.ds(row_a(m(0)), CH), :] = recv_buf[A, 3].astype(jnp.float32)
        rd[B, 3].wait_recv()
        rd[B, 4] = forward(B, 3, 4)
        out_ref[pl.ds(row_b(m(0)), CH), :] = recv_buf[B, 3].astype(jnp.float32)

        rd[A, 4].wait_recv()
        rd[A, 5] = forward(A, 4, 5)
        out_ref[pl.ds(row_a(m(3)), CH), :] = recv_buf[A, 4].astype(jnp.float32)
        rd[B, 4].wait_recv()
        rd[B, 5] = forward(B, 4, 5)
        out_ref[pl.ds(row_b(m(1)), CH), :] = recv_buf[B, 4].astype(jnp.float32)

        rd[A, 5].wait_recv()
        out_ref[pl.ds(row_a(m(2)), CH), :] = recv_buf[A, 5].astype(jnp.float32)
        rd[B, 5].wait_recv()
        out_ref[pl.ds(row_b(m(2)), CH), :] = recv_buf[B, 5].astype(jnp.float32)

        for ring in (A, B):
            for k in (2, 3, 4, 5):
                rd[ring, k].wait_send()

    return pl.pallas_call(
        body,
        out_shape=jax.ShapeDtypeStruct((N_TOK, D_OUT), jnp.float32),
        in_specs=[
            pl.BlockSpec(memory_space=pltpu.VMEM),
            pl.BlockSpec(memory_space=pltpu.VMEM),
            pl.BlockSpec(memory_space=pltpu.VMEM),
        ],
        out_specs=pl.BlockSpec(memory_space=pltpu.VMEM),
        scratch_shapes=[
            pltpu.VMEM((E_LOCAL, D_IN, D_OUT), jnp.bfloat16),
            pltpu.VMEM((2, 2, CH, D_OUT), jnp.bfloat16),
            pltpu.VMEM((2, N_HOPS, CH, D_OUT), jnp.bfloat16),
            pltpu.SemaphoreType.DMA((2, N_HOPS)),
            pltpu.SemaphoreType.DMA((2, N_HOPS)),
        ],
        compiler_params=pltpu.CompilerParams(collective_id=0),
    )(x, route_idx, expert_W)


# device time: 51511 ns/iter; 1.9580x vs baseline; 1.1620x over previous
import jax
import jax.numpy as jnp
from jax import lax
from jax.experimental import pallas as pl
from jax.experimental.pallas import tpu as pltpu

N_DEV = 4
N_TOK = 2048
D_IN = 512
D_OUT = 1024
E_LOCAL = 4
HALF = N_TOK // 2
CH = HALF // N_DEV
SUB = CH // 2
N_HOPS = 2 * (N_DEV - 1)
A, B = 0, 1


def kernel(x, router_W, route_idx, expert_W):
    del router_W

    def body(x_ref, idx_ref, w_ref, out_ref, wb_ref, stage_ref, recv_buf,
             send_sems, recv_sems):
        my = lax.axis_index("i")
        left = lax.rem(my + N_DEV - 1, N_DEV)
        right = lax.rem(my + 1, N_DEV)

        def m(k):
            return lax.rem(my + k, N_DEV)

        def row_a(c):
            return c * CH

        def row_b(c):
            return HALF + c * CH

        barrier_sem = pltpu.get_barrier_semaphore()
        for nbr in (left, right):
            pl.semaphore_signal(
                barrier_sem, inc=1,
                device_id=(nbr,), device_id_type=pl.DeviceIdType.MESH,
            )
        pl.semaphore_wait(barrier_sem, 2)

        for j in range(E_LOCAL):
            wb_ref[j] = w_ref[j].astype(jnp.bfloat16)

        def compute_chunk(row0):
            xb = x_ref[pl.ds(row0, CH), :].astype(jnp.bfloat16)
            idc = idx_ref[pl.ds(row0, CH), :]
            acc = jnp.zeros((CH, D_OUT), jnp.float32)
            for j in range(E_LOCAL):
                ge = my * E_LOCAL + j
                xm = jnp.where(idc == ge, xb, jnp.bfloat16(0.0))
                acc = acc + lax.dot(
                    xm, wb_ref[j], preferred_element_type=jnp.float32)
            out_ref[pl.ds(row0, CH), :] = acc

        def add_sub(ring, k, row0, u):
            r = pl.ds(row0 + u * SUB, SUB)
            out_ref[r, :] = (
                out_ref[r, :]
                + recv_buf[ring, k, pl.ds(u * SUB, SUB), :].astype(jnp.float32))

        def store_sub(ring, k, row0, u):
            out_ref[pl.ds(row0 + u * SUB, SUB), :] = (
                recv_buf[ring, k, pl.ds(u * SUB, SUB), :].astype(jnp.float32))

        dst_dev = {A: right, B: left}

        def stage_send(ring, k, slot, row0, u):
            stage_ref[ring, slot, pl.ds(u * SUB, SUB), :] = (
                out_ref[pl.ds(row0 + u * SUB, SUB), :].astype(jnp.bfloat16))
            r = pltpu.make_async_remote_copy(
                src_ref=stage_ref.at[ring, slot, pl.ds(u * SUB, SUB), :],
                dst_ref=recv_buf.at[ring, k, pl.ds(u * SUB, SUB), :],
                send_sem=send_sems.at[ring, k, u],
                recv_sem=recv_sems.at[ring, k, u],
                device_id=(dst_dev[ring],),
                device_id_type=pl.DeviceIdType.MESH,
            )
            r.start()
            return r

        def forward_sub(ring, k_src, k, u):
            r = pltpu.make_async_remote_copy(
                src_ref=recv_buf.at[ring, k_src, pl.ds(u * SUB, SUB), :],
                dst_ref=recv_buf.at[ring, k, pl.ds(u * SUB, SUB), :],
                send_sem=send_sems.at[ring, k, u],
                recv_sem=recv_sems.at[ring, k, u],
                device_id=(dst_dev[ring],),
                device_id_type=pl.DeviceIdType.MESH,
            )
            r.start()
            return r

        rd = {}

        compute_chunk(row_a(m(0)))
        for u in (0, 1):
            rd[A, 0, u] = stage_send(A, 0, 0, row_a(m(0)), u)
        compute_chunk(row_b(m(0)))
        for u in (0, 1):
            rd[B, 0, u] = stage_send(B, 0, 0, row_b(m(0)), u)
        compute_chunk(row_a(m(3)))
        compute_chunk(row_b(m(1)))

        for u in (0, 1):
            rd[A, 0, u].wait_recv()
            add_sub(A, 0, row_a(m(3)), u)
            rd[A, 1, u] = stage_send(A, 1, 1, row_a(m(3)), u)
            rd[B, 0, u].wait_recv()
            add_sub(B, 0, row_b(m(1)), u)
            rd[B, 1, u] = stage_send(B, 1, 1, row_b(m(1)), u)
        compute_chunk(row_a(m(2)))
        compute_chunk(row_b(m(2)))

        for ring in (A, B):
            for u in (0, 1):
                rd[ring, 0, u].wait_send()
        for u in (0, 1):
            rd[A, 1, u].wait_recv()
            add_sub(A, 1, row_a(m(2)), u)
            rd[A, 2, u] = stage_send(A, 2, 0, row_a(m(2)), u)
            rd[B, 1, u].wait_recv()
            add_sub(B, 1, row_b(m(2)), u)
            rd[B, 2, u] = stage_send(B, 2, 0, row_b(m(2)), u)
        compute_chunk(row_a(m(1)))
        compute_chunk(row_b(m(3)))

        for ring in (A, B):
            for u in (0, 1):
                rd[ring, 1, u].wait_send()
        for u in (0, 1):
            rd[A, 2, u].wait_recv()
            add_sub(A, 2, row_a(m(1)), u)
            rd[A, 3, u] = stage_send(A, 3, 1, row_a(m(1)), u)
            rd[B, 2, u].wait_recv()
            add_sub(B, 2, row_b(m(3)), u)
            rd[B, 3, u] = stage_send(B, 3, 1, row_b(m(3)), u)

        for u in (0, 1):
            rd[A, 3, u].wait_recv()
            rd[A, 4, u] = forward_sub(A, 3, 4, u)
            store_sub(A, 3, row_a(m(0)), u)
            rd[B, 3, u].wait_recv()
            rd[B, 4, u] = forward_sub(B, 3, 4, u)
            store_sub(B, 3, row_b(m(0)), u)

        for u in (0, 1):
            rd[A, 4, u].wait_recv()
            rd[A, 5, u] = forward_sub(A, 4, 5, u)
            store_sub(A, 4, row_a(m(3)), u)
            rd[B, 4, u].wait_recv()
            rd[B, 5, u] = forward_sub(B, 4, 5, u)
            store_sub(B, 4, row_b(m(1)), u)

        for u in (0, 1):
            rd[A, 5, u].wait_recv()
            store_sub(A, 5, row_a(m(2)), u)
            rd[B, 5, u].wait_recv()
            store_sub(B, 5, row_b(m(2)), u)

        for ring in (A, B):
            for k in (2, 3, 4, 5):
                for u in (0, 1):
                    rd[ring, k, u].wait_send()

    return pl.pallas_call(
        body,
        out_shape=jax.ShapeDtypeStruct((N_TOK, D_OUT), jnp.float32),
        in_specs=[
            pl.BlockSpec(memory_space=pltpu.VMEM),
            pl.BlockSpec(memory_space=pltpu.VMEM),
            pl.BlockSpec(memory_space=pltpu.VMEM),
        ],
        out_specs=pl.BlockSpec(memory_space=pltpu.VMEM),
        scratch_shapes=[
            pltpu.VMEM((E_LOCAL, D_IN, D_OUT), jnp.bfloat16),
            pltpu.VMEM((2, 2, CH, D_OUT), jnp.bfloat16),
            pltpu.VMEM((2, N_HOPS, CH, D_OUT), jnp.bfloat16),
            pltpu.SemaphoreType.DMA((2, N_HOPS, 2)),
            pltpu.SemaphoreType.DMA((2, N_HOPS, 2)),
        ],
        compiler_params=pltpu.CompilerParams(collective_id=0),
    )(x, route_idx, expert_W)


# device time: 50346 ns/iter; 2.0033x vs baseline; 1.0231x over previous
import jax
import jax.numpy as jnp
from jax import lax
from jax.experimental import pallas as pl
from jax.experimental.pallas import tpu as pltpu

N_DEV = 4
N_TOK = 2048
D_IN = 512
D_OUT = 1024
E_LOCAL = 4
HALF = N_TOK // 2
CH = HALF // N_DEV
SUB = CH // 2
N_HOPS = 2 * (N_DEV - 1)
A, B = 0, 1


def kernel(x, router_W, route_idx, expert_W):
    del router_W

    def body(x_ref, idx_ref, w_ref, out_ref, wb_ref, loc_ref, recv_buf,
             send_sems, recv_sems):
        my = lax.axis_index("i")
        left = lax.rem(my + N_DEV - 1, N_DEV)
        right = lax.rem(my + 1, N_DEV)

        def m(k):
            return lax.rem(my + k, N_DEV)

        def row_a(c):
            return c * CH

        def row_b(c):
            return HALF + c * CH

        chunk_rows = {
            (A, 0): row_a(m(0)), (A, 1): row_a(m(3)),
            (A, 2): row_a(m(2)), (A, 3): row_a(m(1)),
            (B, 0): row_b(m(0)), (B, 1): row_b(m(1)),
            (B, 2): row_b(m(2)), (B, 3): row_b(m(3)),
        }

        barrier_sem = pltpu.get_barrier_semaphore()
        for nbr in (left, right):
            pl.semaphore_signal(
                barrier_sem, inc=1,
                device_id=(nbr,), device_id_type=pl.DeviceIdType.MESH,
            )
        pl.semaphore_wait(barrier_sem, 2)

        for j in range(E_LOCAL):
            wb_ref[j] = w_ref[j].astype(jnp.bfloat16)

        def compute_chunk(ring, p):
            row0 = chunk_rows[ring, p]
            xb = x_ref[pl.ds(row0, CH), :].astype(jnp.bfloat16)
            idc = idx_ref[pl.ds(row0, CH), :]
            acc = jnp.zeros((CH, D_OUT), jnp.float32)
            for j in range(E_LOCAL):
                ge = my * E_LOCAL + j
                xm = jnp.where(idc == ge, xb, jnp.bfloat16(0.0))
                acc = acc + lax.dot(
                    xm, wb_ref[j], preferred_element_type=jnp.float32)
            loc_ref[ring, p] = acc.astype(jnp.bfloat16)

        def send_sub(ring, k, p, u):
            r = pltpu.make_async_remote_copy(
                src_ref=loc_ref.at[ring, p, pl.ds(u * SUB, SUB), :],
                dst_ref=recv_buf.at[ring, k, pl.ds(u * SUB, SUB), :],
                send_sem=send_sems.at[ring, k, u],
                recv_sem=recv_sems.at[ring, k, u],
                device_id=(right if ring == A else left,),
                device_id_type=pl.DeviceIdType.MESH,
            )
            r.start()
            return r

        def forward_sub(ring, k_src, k, u):
            r = pltpu.make_async_remote_copy(
                src_ref=recv_buf.at[ring, k_src, pl.ds(u * SUB, SUB), :],
                dst_ref=recv_buf.at[ring, k, pl.ds(u * SUB, SUB), :],
                send_sem=send_sems.at[ring, k, u],
                recv_sem=recv_sems.at[ring, k, u],
                device_id=(right if ring == A else left,),
                device_id_type=pl.DeviceIdType.MESH,
            )
            r.start()
            return r

        def add_sub(ring, k, p, u):
            s = pl.ds(u * SUB, SUB)
            loc_ref[ring, p, s, :] = (
                loc_ref[ring, p, s, :] + recv_buf[ring, k, s, :])

        def store_sub(ring, k, row0, u):
            out_ref[pl.ds(row0 + u * SUB, SUB), :] = (
                recv_buf[ring, k, pl.ds(u * SUB, SUB), :].astype(jnp.float32))

        rd = {}

        compute_chunk(A, 0)
        for u in (0, 1):
            rd[A, 0, u] = send_sub(A, 0, 0, u)
        compute_chunk(B, 0)
        for u in (0, 1):
            rd[B, 0, u] = send_sub(B, 0, 0, u)
        compute_chunk(A, 1)
        compute_chunk(B, 1)

        for s in (0, 1):
            for u in (0, 1):
                rd[A, s, u].wait_recv()
                add_sub(A, s, s + 1, u)
                rd[A, s + 1, u] = send_sub(A, s + 1, s + 1, u)
                rd[B, s, u].wait_recv()
                add_sub(B, s, s + 1, u)
                rd[B, s + 1, u] = send_sub(B, s + 1, s + 1, u)
            compute_chunk(A, s + 2)
            compute_chunk(B, s + 2)

        for u in (0, 1):
            for ring, row0 in ((A, row_a(m(1))), (B, row_b(m(3)))):
                rd[ring, 2, u].wait_recv()
                s_ = pl.ds(u * SUB, SUB)
                summed = (loc_ref[ring, 3, s_, :].astype(jnp.float32)
                          + recv_buf[ring, 2, s_, :].astype(jnp.float32))
                loc_ref[ring, 3, s_, :] = summed.astype(jnp.bfloat16)
                rd[ring, 3, u] = send_sub(ring, 3, 3, u)
                out_ref[pl.ds(row0 + u * SUB, SUB), :] = summed

        for u in (0, 1):
            rd[A, 3, u].wait_recv()
            rd[A, 4, u] = forward_sub(A, 3, 4, u)
            store_sub(A, 3, row_a(m(0)), u)
            rd[B, 3, u].wait_recv()
            rd[B, 4, u] = forward_sub(B, 3, 4, u)
            store_sub(B, 3, row_b(m(0)), u)

        for u in (0, 1):
            rd[A, 4, u].wait_recv()
            rd[A, 5, u] = forward_sub(A, 4, 5, u)
            store_sub(A, 4, row_a(m(3)), u)
            rd[B, 4, u].wait_recv()
            rd[B, 5, u] = forward_sub(B, 4, 5, u)
            store_sub(B, 4, row_b(m(1)), u)

        for u in (0, 1):
            rd[A, 5, u].wait_recv()
            store_sub(A, 5, row_a(m(2)), u)
            rd[B, 5, u].wait_recv()
            store_sub(B, 5, row_b(m(2)), u)

        for ring in (A, B):
            for k in range(N_HOPS):
                for u in (0, 1):
                    rd[ring, k, u].wait_send()

    return pl.pallas_call(
        body,
        out_shape=jax.ShapeDtypeStruct((N_TOK, D_OUT), jnp.float32),
        in_specs=[
            pl.BlockSpec(memory_space=pltpu.VMEM),
            pl.BlockSpec(memory_space=pltpu.VMEM),
            pl.BlockSpec(memory_space=pltpu.VMEM),
        ],
        out_specs=pl.BlockSpec(memory_space=pltpu.VMEM),
        scratch_shapes=[
            pltpu.VMEM((E_LOCAL, D_IN, D_OUT), jnp.bfloat16),
            pltpu.VMEM((2, N_DEV, CH, D_OUT), jnp.bfloat16),
            pltpu.VMEM((2, N_HOPS, CH, D_OUT), jnp.bfloat16),
            pltpu.SemaphoreType.DMA((2, N_HOPS, 2)),
            pltpu.SemaphoreType.DMA((2, N_HOPS, 2)),
        ],
        compiler_params=pltpu.CompilerParams(collective_id=0),
    )(x, route_idx, expert_W)
